# baseline (speedup 1.0000x reference)
import functools

import jax
import jax.numpy as jnp
from jax import lax
from jax.experimental import pallas as pl
from jax.experimental.pallas import tpu as pltpu
from jax.experimental.pallas import tpu_sc as plsc

N = 4096
D = 2048
E = 65536
H = 64

F32 = jnp.float32

NC, NS = 2, 16
NW = NC * NS
CH = 128
EPW = E // NW
NCHUNK = EPW // CH
TRASH = N
NR = N + 128
ZR = NR // NS
WR = N // NS
FB = 128
FD = 128

BM = 512


def _normalize_rows(x):
    n = jnp.sqrt(jnp.sum(x * x, axis=1, keepdims=True))
    return x / jnp.maximum(n, 1e-12)


def _project_body(x_ref, w_ref, gp_ref, gn_ref, ys_ref):
    p = jnp.dot(x_ref[...], w_ref[...], preferred_element_type=F32)
    ones = jnp.ones((BM, FB - H), F32)
    gp_ref[...] = jnp.concatenate([p[:, :H], ones], axis=1)
    gn_ref[...] = jnp.concatenate([p[:, H:2 * H], ones], axis=1)
    ys_ref[...] = p[:, 2 * H:]


def _project(X, Wcat):
    grid = (N // BM,)
    return pl.pallas_call(
        _project_body,
        grid=grid,
        in_specs=[
            pl.BlockSpec((BM, D), lambda i: (i, 0)),
            pl.BlockSpec((D, 4 * H), lambda i: (0, 0)),
        ],
        out_specs=[
            pl.BlockSpec((BM, FB), lambda i: (i, 0)),
            pl.BlockSpec((BM, FB), lambda i: (i, 0)),
            pl.BlockSpec((BM, 2 * H), lambda i: (i, 0)),
        ],
        out_shape=[
            jax.ShapeDtypeStruct((N, FB), F32),
            jax.ShapeDtypeStruct((N, FB), F32),
            jax.ShapeDtypeStruct((N, 2 * H), F32),
        ],
    )(X, Wcat)


NB = 2
NG = NCHUNK // NB


def _seg_sum_set(eref, gref, acc, wid, idx2, radja, gbuf, gsem, ssem):
    ebase = wid * EPW
    pltpu.sync_copy(eref.at[:, pl.ds(ebase, EPW)], idx2)
    pltpu.async_copy(gref.at[idx2.at[1, pl.ds(0, CH)]],
                     gbuf.at[0], gsem.at[0])

    def adj(j, carry):
        r = idx2[0, pl.ds(j * 16, 16)]
        cc = idx2[1, pl.ds(j * 16, 16)]
        radja[pl.ds(j * 16, 16)] = jnp.where(r == cc, TRASH, r)
        return carry

    lax.fori_loop(0, EPW // 16, adj, 0)

    def ring(g, carry):
        for b in range(NB):
            k = g * NB + b
            b2 = (b + 1) % NB
            rows = radja.at[pl.ds(k * CH, CH)]
            pltpu.make_async_copy(
                gref.at[idx2.at[1, pl.ds(k * CH, CH)]],
                gbuf.at[b], gsem.at[b]).wait()
            pltpu.async_copy(gbuf.at[b], acc.at[rows], ssem.at[b],
                             add=True)

            @pl.when(k >= 1)
            def _drain(b2=b2, k=k):
                pltpu.make_async_copy(
                    gbuf.at[b2], acc.at[radja.at[pl.ds((k - 1) * CH, CH)]],
                    ssem.at[b2]).wait()

            @pl.when(k + 1 < NCHUNK)
            def _issue(k=k, b2=b2):
                pltpu.async_copy(
                    gref.at[idx2.at[1, pl.ds((k + 1) * CH, CH)]],
                    gbuf.at[b2], gsem.at[b2])
        return carry

    lax.fori_loop(0, NG, ring, 0)
    last = (NCHUNK - 1) % NB
    pltpu.make_async_copy(
        gbuf.at[last], acc.at[radja.at[pl.ds((NCHUNK - 1) * CH, CH)]],
        ssem.at[last]).wait()


def _sc_base_body(gp, gn, ep, en, z80, accp_out, accn_out,
                  idx2, radja, gbuf, accp, accn, gsem, ssem):
    c = lax.axis_index("c")
    s = lax.axis_index("s")
    wid = s * NC + c
    pltpu.sync_copy(z80, accp.at[pl.ds(s * ZR, ZR)])
    pltpu.sync_copy(z80, accn.at[pl.ds(s * ZR, ZR)])
    plsc.subcore_barrier()

    _seg_sum_set(ep, gp, accp, wid, idx2, radja, gbuf, gsem, ssem)
    _seg_sum_set(en, gn, accn, wid, idx2, radja, gbuf, gsem, ssem)

    plsc.subcore_barrier()
    pltpu.sync_copy(accp.at[pl.ds(s * WR, WR)],
                    accp_out.at[c, pl.ds(s * WR, WR)])
    pltpu.sync_copy(accn.at[pl.ds(s * WR, WR)],
                    accn_out.at[c, pl.ds(s * WR, WR)])


def _sc_base(gp, gn, ep, en):
    z80 = jnp.zeros((ZR, FB), F32)
    mesh = plsc.VectorSubcoreMesh(core_axis_name="c", subcore_axis_name="s")
    fn = functools.partial(
        pl.kernel,
        mesh=mesh,
        out_type=[
            jax.ShapeDtypeStruct((NC, N, FB), F32),
            jax.ShapeDtypeStruct((NC, N, FB), F32),
        ],
        scratch_types=[
            pltpu.VMEM((2, EPW), jnp.int32),
            pltpu.VMEM((EPW,), jnp.int32),
            pltpu.VMEM((NB, CH, FB), F32),
            pltpu.VMEM_SHARED((NR, FB), F32),
            pltpu.VMEM_SHARED((NR, FB), F32),
            pltpu.SemaphoreType.DMA((NB,)),
            pltpu.SemaphoreType.DMA((NB,)),
        ],
    )(_sc_base_body)
    return fn(gp, gn, ep, en, z80)


def _combine_body(ap_ref, an_ref, ys_ref, bp_ref, bn_ref,
                  hcat_ref, invp_ref, invn_ref):
    ap = ap_ref[0] + ap_ref[1]
    an = an_ref[0] + an_ref[1]
    cp = ap[:, H:H + 1]
    cn = an[:, H:H + 1]
    hp = ap[:, :H] / jnp.maximum(cp, 1.0) + ys_ref[:, :H] + bp_ref[...]
    hn = an[:, :H] / jnp.maximum(cn, 1.0) + ys_ref[:, H:] + bn_ref[...]
    hp = jnp.tanh(_normalize_rows(hp))
    hn = jnp.tanh(_normalize_rows(hn))
    hcat_ref[...] = jnp.concatenate([hp, hn], axis=1)
    invp_ref[...] = jnp.broadcast_to(1.0 / (cp + 1.0), (BM, FD))
    invn_ref[...] = jnp.broadcast_to(1.0 / (cn + 1.0), (BM, FD))


def _combine(accp, accn, ys, bpb, bnb):
    grid = (N // BM,)
    return pl.pallas_call(
        _combine_body,
        grid=grid,
        in_specs=[
            pl.BlockSpec((NC, BM, FB), lambda i: (0, i, 0)),
            pl.BlockSpec((NC, BM, FB), lambda i: (0, i, 0)),
            pl.BlockSpec((BM, 2 * H), lambda i: (i, 0)),
            pl.BlockSpec((1, H), lambda i: (0, 0)),
            pl.BlockSpec((1, H), lambda i: (0, 0)),
        ],
        out_specs=[
            pl.BlockSpec((BM, FD), lambda i: (i, 0)),
            pl.BlockSpec((BM, FD), lambda i: (i, 0)),
            pl.BlockSpec((BM, FD), lambda i: (i, 0)),
        ],
        out_shape=[
            jax.ShapeDtypeStruct((N, FD), F32),
            jax.ShapeDtypeStruct((N, FD), F32),
            jax.ShapeDtypeStruct((N, FD), F32),
        ],
    )(accp, accn, ys, bpb.reshape(1, H), bnb.reshape(1, H))


def _sc_deep_body(hcat, ep, en, z128, tp_out, tn_out,
                  idx2, radja, gbuf, accp, accn, gsem, ssem):
    c = lax.axis_index("c")
    s = lax.axis_index("s")
    wid = s * NC + c
    pltpu.sync_copy(z128, accp.at[pl.ds(s * ZR, ZR)])
    pltpu.sync_copy(z128, accn.at[pl.ds(s * ZR, ZR)])
    plsc.subcore_barrier()

    _seg_sum_set(ep, hcat, accp, wid, idx2, radja, gbuf, gsem, ssem)
    _seg_sum_set(en, hcat, accn, wid, idx2, radja, gbuf, gsem, ssem)

    plsc.subcore_barrier()
    pltpu.sync_copy(accp.at[pl.ds(s * WR, WR)],
                    tp_out.at[c, pl.ds(s * WR, WR)])
    pltpu.sync_copy(accn.at[pl.ds(s * WR, WR)],
                    tn_out.at[c, pl.ds(s * WR, WR)])


def _sc_deep(hcat, ep, en):
    z128 = jnp.zeros((ZR, FD), F32)
    mesh = plsc.VectorSubcoreMesh(core_axis_name="c", subcore_axis_name="s")
    fn = functools.partial(
        pl.kernel,
        mesh=mesh,
        out_type=[
            jax.ShapeDtypeStruct((NC, N, FD), F32),
            jax.ShapeDtypeStruct((NC, N, FD), F32),
        ],
        scratch_types=[
            pltpu.VMEM((2, EPW), jnp.int32),
            pltpu.VMEM((EPW,), jnp.int32),
            pltpu.VMEM((NB, CH, FD), F32),
            pltpu.VMEM_SHARED((NR, FD), F32),
            pltpu.VMEM_SHARED((NR, FD), F32),
            pltpu.SemaphoreType.DMA((NB,)),
            pltpu.SemaphoreType.DMA((NB,)),
        ],
    )(_sc_deep_body)
    return fn(hcat, ep, en, z128)


def _deep_combine_body(tp_ref, tn_ref, hcat_ref, invp_ref, invn_ref,
                       wp_ref, wn_ref, bp_ref, bn_ref, xmol_ref):
    hcat = hcat_ref[...]
    up = (tp_ref[0] + tp_ref[1] + hcat) * invp_ref[...]
    un = (tn_ref[0] + tn_ref[1] + hcat) * invn_ref[...]
    hp0 = hcat[:, :H]
    hn0 = hcat[:, H:]
    catp = jnp.concatenate([up[:, :H], un[:, H:], hp0], axis=1)
    catn = jnp.concatenate([up[:, H:], un[:, :H], hn0], axis=1)
    hp_pre = jnp.dot(catp, wp_ref[...], preferred_element_type=F32) + bp_ref[...]
    hn_pre = jnp.dot(catn, wn_ref[...], preferred_element_type=F32) + bn_ref[...]
    hp1 = jnp.tanh(_normalize_rows(hp_pre))
    hn1 = jnp.tanh(_normalize_rows(hn_pre))
    xmol_ref[...] = _normalize_rows(jnp.concatenate([hp1, hn1], axis=1))


def _deep_combine(tp, tn, hcat, invp, invn, Wpd, Wnd, bpd, bnd):
    grid = (N // BM,)
    return pl.pallas_call(
        _deep_combine_body,
        grid=grid,
        in_specs=[
            pl.BlockSpec((NC, BM, FD), lambda i: (0, i, 0)),
            pl.BlockSpec((NC, BM, FD), lambda i: (0, i, 0)),
            pl.BlockSpec((BM, FD), lambda i: (i, 0)),
            pl.BlockSpec((BM, FD), lambda i: (i, 0)),
            pl.BlockSpec((BM, FD), lambda i: (i, 0)),
            pl.BlockSpec((3 * H, H), lambda i: (0, 0)),
            pl.BlockSpec((3 * H, H), lambda i: (0, 0)),
            pl.BlockSpec((1, H), lambda i: (0, 0)),
            pl.BlockSpec((1, H), lambda i: (0, 0)),
        ],
        out_specs=pl.BlockSpec((BM, FD), lambda i: (i, 0)),
        out_shape=jax.ShapeDtypeStruct((N, FD), F32),
    )(tp, tn, hcat, invp, invn, Wpd, Wnd,
      bpd.reshape(1, H), bnd.reshape(1, H))


BP = 512
GN_ = N // BP
RPB = BP * N // 128


def _pred_body(xi_ref, xall_ref, mask_ref, lab_ref, pred_ref, loss_ref):
    i = pl.program_id(0)

    @pl.when(i == 0)
    def _init():
        loss_ref[...] = jnp.zeros((1, 1), F32)

    b = lax.dot_general(xi_ref[...], xall_ref[...],
                        (((1,), (1,)), ((), ())),
                        preferred_element_type=F32) * mask_ref[...]
    b8 = b.reshape(RPB, 128)
    pred_ref[...] = b8
    r = b8 - lab_ref[...]
    loss_ref[...] += jnp.sum(r * r).reshape(1, 1)

    @pl.when(i == GN_ - 1)
    def _fin():
        loss_ref[...] = loss_ref[...] * (1.0 / float(N * N))


def _pred_loss(xmol, label_mask, labels8):
    grid = (GN_,)
    return pl.pallas_call(
        _pred_body,
        grid=grid,
        in_specs=[
            pl.BlockSpec((BP, FD), lambda i: (i, 0)),
            pl.BlockSpec((N, FD), lambda i: (0, 0)),
            pl.BlockSpec((BP, N), lambda i: (i, 0)),
            pl.BlockSpec((RPB, 128), lambda i: (i, 0)),
        ],
        out_specs=[
            pl.BlockSpec((RPB, 128), lambda i: (i, 0)),
            pl.BlockSpec((1, 1), lambda i: (0, 0)),
        ],
        out_shape=[
            jax.ShapeDtypeStruct((N * N // 128, 128), F32),
            jax.ShapeDtypeStruct((1, 1), F32),
        ],
    )(xmol, xmol, label_mask, labels8)


def kernel(X, positive_edges, negative_edges, labels, label_mask,
           Wpb, bpb, Wnb, bnb, Wpd, bpd, Wnd, bnd):
    ep = positive_edges.astype(jnp.int32)
    en = negative_edges.astype(jnp.int32)
    Wcat = jnp.concatenate([Wpb[:D], Wnb[:D], Wpb[D:], Wnb[D:]], axis=1)

    gp, gn, ys = _project(X, Wcat)
    accp, accn = _sc_base(gp, gn, ep, en)
    hcat, invp, invn = _combine(accp, accn, ys, bpb, bnb)
    tp, tn = _sc_deep(hcat, ep, en)
    xmol = _deep_combine(tp, tn, hcat, invp, invn, Wpd, Wnd, bpd, bnd)
    pred2, lossm = _pred_loss(xmol, label_mask, labels.reshape(N * N // 128, 128))
    return (lossm[0, 0], xmol, pred2.reshape(-1))


# speedup vs baseline: 1.0195x; 1.0195x over previous
import functools

import jax
import jax.numpy as jnp
from jax import lax
from jax.experimental import pallas as pl
from jax.experimental.pallas import tpu as pltpu
from jax.experimental.pallas import tpu_sc as plsc

N = 4096
D = 2048
E = 65536
H = 64

F32 = jnp.float32

NC, NS = 2, 16
NW = NC * NS
CH = 128
EPW = E // NW
NCHUNK = EPW // CH
TRASH = N
NR = N + 128
ZR = NR // NS
WR = N // NS
FB = 128
FD = 128

BM = 512


def _normalize_rows(x):
    n = jnp.sqrt(jnp.sum(x * x, axis=1, keepdims=True))
    return x / jnp.maximum(n, 1e-12)


def _project_body(x_ref, wpt_ref, wnt_ref, wpb_ref, wnb_ref,
                  gp_ref, gn_ref, ys_ref):
    x = x_ref[...]
    ones = jnp.ones((BM, FB - H), F32)
    pp = jnp.dot(x, wpt_ref[...], preferred_element_type=F32)
    pn = jnp.dot(x, wnt_ref[...], preferred_element_type=F32)
    gp_ref[...] = jnp.concatenate([pp, ones], axis=1)
    gn_ref[...] = jnp.concatenate([pn, ones], axis=1)
    sp = jnp.dot(x, wpb_ref[...], preferred_element_type=F32)
    sn = jnp.dot(x, wnb_ref[...], preferred_element_type=F32)
    ys_ref[...] = jnp.concatenate([sp, sn], axis=1)


def _project(X, Wpb, Wnb):
    grid = (N // BM,)
    return pl.pallas_call(
        _project_body,
        grid=grid,
        in_specs=[
            pl.BlockSpec((BM, D), lambda i: (i, 0)),
            pl.BlockSpec((D, H), lambda i: (0, 0)),
            pl.BlockSpec((D, H), lambda i: (0, 0)),
            pl.BlockSpec((D, H), lambda i: (1, 0)),
            pl.BlockSpec((D, H), lambda i: (1, 0)),
        ],
        out_specs=[
            pl.BlockSpec((BM, FB), lambda i: (i, 0)),
            pl.BlockSpec((BM, FB), lambda i: (i, 0)),
            pl.BlockSpec((BM, 2 * H), lambda i: (i, 0)),
        ],
        out_shape=[
            jax.ShapeDtypeStruct((N, FB), F32),
            jax.ShapeDtypeStruct((N, FB), F32),
            jax.ShapeDtypeStruct((N, 2 * H), F32),
        ],
    )(X, Wpb, Wnb, Wpb, Wnb)


NB = 2
NG = NCHUNK // NB


def _seg_sum_set(eref, gref, acc, wid, idx2, radja, gbuf, gsem, ssem):
    ebase = wid * EPW
    pltpu.sync_copy(eref.at[:, pl.ds(ebase, EPW)], idx2)
    for b in range(NB):
        pltpu.async_copy(gref.at[idx2.at[1, pl.ds(b * CH, CH)]],
                         gbuf.at[b], gsem.at[b])

    def adj(j, carry):
        r = idx2[0, pl.ds(j * 16, 16)]
        cc = idx2[1, pl.ds(j * 16, 16)]
        radja[pl.ds(j * 16, 16)] = jnp.where(r == cc, TRASH, r)
        return carry

    lax.fori_loop(0, EPW // 16, adj, 0)

    def ring(g, carry):
        for b in range(NB):
            k = g * NB + b
            pltpu.make_async_copy(
                gref.at[idx2.at[1, pl.ds(k * CH, CH)]],
                gbuf.at[b], gsem.at[b]).wait()
            pltpu.sync_copy(gbuf.at[b],
                            acc.at[radja.at[pl.ds(k * CH, CH)]], add=True)

            @pl.when(k + NB < NCHUNK)
            def _issue(k=k, b=b):
                pltpu.async_copy(
                    gref.at[idx2.at[1, pl.ds((k + NB) * CH, CH)]],
                    gbuf.at[b], gsem.at[b])
        return carry

    lax.fori_loop(0, NG, ring, 0)


def _sc_base_body(gp, gn, ep, en, z80, accp_out, accn_out,
                  idx2, radja, gbuf, accp, accn, gsem, ssem):
    c = lax.axis_index("c")
    s = lax.axis_index("s")
    wid = s * NC + c
    pltpu.sync_copy(z80, accp.at[pl.ds(s * ZR, ZR)])
    pltpu.sync_copy(z80, accn.at[pl.ds(s * ZR, ZR)])
    plsc.subcore_barrier()

    _seg_sum_set(ep, gp, accp, wid, idx2, radja, gbuf, gsem, ssem)
    _seg_sum_set(en, gn, accn, wid, idx2, radja, gbuf, gsem, ssem)

    plsc.subcore_barrier()
    pltpu.sync_copy(accp.at[pl.ds(s * WR, WR)],
                    accp_out.at[c, pl.ds(s * WR, WR)])
    pltpu.sync_copy(accn.at[pl.ds(s * WR, WR)],
                    accn_out.at[c, pl.ds(s * WR, WR)])


def _sc_base(gp, gn, ep, en):
    z80 = jnp.zeros((ZR, FB), F32)
    mesh = plsc.VectorSubcoreMesh(core_axis_name="c", subcore_axis_name="s")
    fn = functools.partial(
        pl.kernel,
        mesh=mesh,
        out_type=[
            jax.ShapeDtypeStruct((NC, N, FB), F32),
            jax.ShapeDtypeStruct((NC, N, FB), F32),
        ],
        scratch_types=[
            pltpu.VMEM((2, EPW), jnp.int32),
            pltpu.VMEM((EPW,), jnp.int32),
            pltpu.VMEM((NB, CH, FB), F32),
            pltpu.VMEM_SHARED((NR, FB), F32),
            pltpu.VMEM_SHARED((NR, FB), F32),
            pltpu.SemaphoreType.DMA((NB,)),
            pltpu.SemaphoreType.DMA((NB,)),
        ],
    )(_sc_base_body)
    return fn(gp, gn, ep, en, z80)


def _combine_body(ap_ref, an_ref, ys_ref, bp_ref, bn_ref,
                  hcat_ref, inv_ref):
    ap = ap_ref[0] + ap_ref[1]
    an = an_ref[0] + an_ref[1]
    cp = ap[:, H:H + 1]
    cn = an[:, H:H + 1]
    hp = ap[:, :H] / jnp.maximum(cp, 1.0) + ys_ref[:, :H] + bp_ref[...]
    hn = an[:, :H] / jnp.maximum(cn, 1.0) + ys_ref[:, H:] + bn_ref[...]
    hp = jnp.tanh(_normalize_rows(hp))
    hn = jnp.tanh(_normalize_rows(hn))
    hcat_ref[...] = jnp.concatenate([hp, hn], axis=1)
    inv_ref[...] = jnp.concatenate(
        [jnp.broadcast_to(1.0 / (cp + 1.0), (BM, H)),
         jnp.broadcast_to(1.0 / (cn + 1.0), (BM, H))], axis=1)


def _combine(accp, accn, ys, bpb, bnb):
    grid = (N // BM,)
    return pl.pallas_call(
        _combine_body,
        grid=grid,
        in_specs=[
            pl.BlockSpec((NC, BM, FB), lambda i: (0, i, 0)),
            pl.BlockSpec((NC, BM, FB), lambda i: (0, i, 0)),
            pl.BlockSpec((BM, 2 * H), lambda i: (i, 0)),
            pl.BlockSpec((1, H), lambda i: (0, 0)),
            pl.BlockSpec((1, H), lambda i: (0, 0)),
        ],
        out_specs=[
            pl.BlockSpec((BM, FD), lambda i: (i, 0)),
            pl.BlockSpec((BM, FD), lambda i: (i, 0)),
        ],
        out_shape=[
            jax.ShapeDtypeStruct((N, FD), F32),
            jax.ShapeDtypeStruct((N, FD), F32),
        ],
    )(accp, accn, ys, bpb.reshape(1, H), bnb.reshape(1, H))


def _sc_deep_body(hcat, ep, en, z128, tp_out, tn_out,
                  idx2, radja, gbuf, accp, accn, gsem, ssem):
    c = lax.axis_index("c")
    s = lax.axis_index("s")
    wid = s * NC + c
    pltpu.sync_copy(z128, accp.at[pl.ds(s * ZR, ZR)])
    pltpu.sync_copy(z128, accn.at[pl.ds(s * ZR, ZR)])
    plsc.subcore_barrier()

    _seg_sum_set(ep, hcat, accp, wid, idx2, radja, gbuf, gsem, ssem)
    _seg_sum_set(en, hcat, accn, wid, idx2, radja, gbuf, gsem, ssem)

    plsc.subcore_barrier()
    pltpu.sync_copy(accp.at[pl.ds(s * WR, WR)],
                    tp_out.at[c, pl.ds(s * WR, WR)])
    pltpu.sync_copy(accn.at[pl.ds(s * WR, WR)],
                    tn_out.at[c, pl.ds(s * WR, WR)])


def _sc_deep(hcat, ep, en):
    z128 = jnp.zeros((ZR, FD), F32)
    mesh = plsc.VectorSubcoreMesh(core_axis_name="c", subcore_axis_name="s")
    fn = functools.partial(
        pl.kernel,
        mesh=mesh,
        out_type=[
            jax.ShapeDtypeStruct((NC, N, FD), F32),
            jax.ShapeDtypeStruct((NC, N, FD), F32),
        ],
        scratch_types=[
            pltpu.VMEM((2, EPW), jnp.int32),
            pltpu.VMEM((EPW,), jnp.int32),
            pltpu.VMEM((NB, CH, FD), F32),
            pltpu.VMEM_SHARED((NR, FD), F32),
            pltpu.VMEM_SHARED((NR, FD), F32),
            pltpu.SemaphoreType.DMA((NB,)),
            pltpu.SemaphoreType.DMA((NB,)),
        ],
    )(_sc_deep_body)
    return fn(hcat, ep, en, z128)


def _deep_combine_body(tp_ref, tn_ref, hcat_ref, inv_ref,
                       wp_ref, wn_ref, bp_ref, bn_ref, xmol_ref):
    hcat = hcat_ref[...]
    ip = inv_ref[:, :H]
    iv = inv_ref[:, H:]
    tph = tp_ref[0] + tp_ref[1] + hcat
    tnh = tn_ref[0] + tn_ref[1] + hcat
    hp0 = hcat[:, :H]
    hn0 = hcat[:, H:]
    catp = jnp.concatenate([tph[:, :H] * ip, tnh[:, H:] * iv, hp0], axis=1)
    catn = jnp.concatenate([tph[:, H:] * ip, tnh[:, :H] * iv, hn0], axis=1)
    hp_pre = jnp.dot(catp, wp_ref[...], preferred_element_type=F32) + bp_ref[...]
    hn_pre = jnp.dot(catn, wn_ref[...], preferred_element_type=F32) + bn_ref[...]
    hp1 = jnp.tanh(_normalize_rows(hp_pre))
    hn1 = jnp.tanh(_normalize_rows(hn_pre))
    xmol_ref[...] = _normalize_rows(jnp.concatenate([hp1, hn1], axis=1))


def _deep_combine(tp, tn, hcat, inv, Wpd, Wnd, bpd, bnd):
    grid = (N // BM,)
    return pl.pallas_call(
        _deep_combine_body,
        grid=grid,
        in_specs=[
            pl.BlockSpec((NC, BM, FD), lambda i: (0, i, 0)),
            pl.BlockSpec((NC, BM, FD), lambda i: (0, i, 0)),
            pl.BlockSpec((BM, FD), lambda i: (i, 0)),
            pl.BlockSpec((BM, FD), lambda i: (i, 0)),
            pl.BlockSpec((3 * H, H), lambda i: (0, 0)),
            pl.BlockSpec((3 * H, H), lambda i: (0, 0)),
            pl.BlockSpec((1, H), lambda i: (0, 0)),
            pl.BlockSpec((1, H), lambda i: (0, 0)),
        ],
        out_specs=pl.BlockSpec((BM, FD), lambda i: (i, 0)),
        out_shape=jax.ShapeDtypeStruct((N, FD), F32),
    )(tp, tn, hcat, inv, Wpd, Wnd,
      bpd.reshape(1, H), bnd.reshape(1, H))


BP = 512
GN_ = N // BP
RPB = BP * N // 128


def _pred_body(xi_ref, xall_ref, mask_ref, lab_ref, pred_ref, loss_ref):
    i = pl.program_id(0)

    @pl.when(i == 0)
    def _init():
        loss_ref[...] = jnp.zeros((1, 1), F32)

    b = lax.dot_general(xi_ref[...], xall_ref[...],
                        (((1,), (1,)), ((), ())),
                        preferred_element_type=F32) * mask_ref[...]
    b8 = b.reshape(RPB, 128)
    pred_ref[...] = b8
    r = b8 - lab_ref[...]
    loss_ref[...] += jnp.sum(r * r).reshape(1, 1)

    @pl.when(i == GN_ - 1)
    def _fin():
        loss_ref[...] = loss_ref[...] * (1.0 / float(N * N))


def _pred_loss(xmol, label_mask, labels8):
    grid = (GN_,)
    return pl.pallas_call(
        _pred_body,
        grid=grid,
        in_specs=[
            pl.BlockSpec((BP, FD), lambda i: (i, 0)),
            pl.BlockSpec((N, FD), lambda i: (0, 0)),
            pl.BlockSpec((BP, N), lambda i: (i, 0)),
            pl.BlockSpec((RPB, 128), lambda i: (i, 0)),
        ],
        out_specs=[
            pl.BlockSpec((RPB, 128), lambda i: (i, 0)),
            pl.BlockSpec((1, 1), lambda i: (0, 0)),
        ],
        out_shape=[
            jax.ShapeDtypeStruct((N * N // 128, 128), F32),
            jax.ShapeDtypeStruct((1, 1), F32),
        ],
    )(xmol, xmol, label_mask, labels8)


def kernel(X, positive_edges, negative_edges, labels, label_mask,
           Wpb, bpb, Wnb, bnb, Wpd, bpd, Wnd, bnd):
    ep = positive_edges.astype(jnp.int32)
    en = negative_edges.astype(jnp.int32)

    gp, gn, ys = _project(X, Wpb, Wnb)
    accp, accn = _sc_base(gp, gn, ep, en)
    hcat, inv = _combine(accp, accn, ys, bpb, bnb)
    tp, tn = _sc_deep(hcat, ep, en)
    xmol = _deep_combine(tp, tn, hcat, inv, Wpd, Wnd, bpd, bnd)
    pred2, lossm = _pred_loss(xmol, label_mask, labels.reshape(N * N // 128, 128))
    return (lossm[0, 0], xmol, pred2.reshape(-1))


# speedup vs baseline: 1.0586x; 1.0383x over previous
import functools

import jax
import jax.numpy as jnp
from jax import lax
from jax.experimental import pallas as pl
from jax.experimental.pallas import tpu as pltpu
from jax.experimental.pallas import tpu_sc as plsc

N = 4096
D = 2048
E = 65536
H = 64

F32 = jnp.float32

NC, NS = 2, 16
NW = NC * NS
CH = 128
EPW = E // NW
NCHUNK = EPW // CH
TRASH = N
NR = N + 128
ZR = NR // NS
WR = N // NS
FB = 128
FD = 128

BM = 512


def _normalize_rows(x):
    n = jnp.sqrt(jnp.sum(x * x, axis=1, keepdims=True))
    return x / jnp.maximum(n, 1e-12)


def _project_body(x_ref, w_ref, gp_ref, gn_ref, ys_ref):
    p = jnp.dot(x_ref[...], w_ref[...], preferred_element_type=F32)
    ones = jnp.ones((BM, FB - H), F32)
    gp_ref[...] = jnp.concatenate([p[:, :H], ones], axis=1)
    gn_ref[...] = jnp.concatenate([p[:, H:2 * H], ones], axis=1)
    ys_ref[...] = p[:, 2 * H:]


def _project(X, Wcat):
    grid = (N // BM,)
    return pl.pallas_call(
        _project_body,
        grid=grid,
        in_specs=[
            pl.BlockSpec((BM, D), lambda i: (i, 0)),
            pl.BlockSpec((D, 4 * H), lambda i: (0, 0)),
        ],
        out_specs=[
            pl.BlockSpec((BM, FB), lambda i: (i, 0)),
            pl.BlockSpec((BM, FB), lambda i: (i, 0)),
            pl.BlockSpec((BM, 2 * H), lambda i: (i, 0)),
        ],
        out_shape=[
            jax.ShapeDtypeStruct((N, FB), F32),
            jax.ShapeDtypeStruct((N, FB), F32),
            jax.ShapeDtypeStruct((N, 2 * H), F32),
        ],
    )(X, Wcat)


NB = 2
NG = NCHUNK // NB


def _seg_sum_set(eref, gref, acc, wid, idx2, radja, gbuf, gsem, ssem):
    ebase = wid * EPW
    pltpu.sync_copy(eref.at[:, pl.ds(ebase, EPW)], idx2)
    for b in range(NB):
        pltpu.async_copy(gref.at[idx2.at[1, pl.ds(b * CH, CH)]],
                         gbuf.at[b], gsem.at[b])

    def adj(j, carry):
        r = idx2[0, pl.ds(j * 16, 16)]
        cc = idx2[1, pl.ds(j * 16, 16)]
        radja[pl.ds(j * 16, 16)] = jnp.where(r == cc, TRASH, r)
        return carry

    lax.fori_loop(0, EPW // 16, adj, 0)

    def ring(g, carry):
        for b in range(NB):
            k = g * NB + b
            pltpu.make_async_copy(
                gref.at[idx2.at[1, pl.ds(k * CH, CH)]],
                gbuf.at[b], gsem.at[b]).wait()
            pltpu.sync_copy(gbuf.at[b],
                            acc.at[radja.at[pl.ds(k * CH, CH)]], add=True)

            @pl.when(k + NB < NCHUNK)
            def _issue(k=k, b=b):
                pltpu.async_copy(
                    gref.at[idx2.at[1, pl.ds((k + NB) * CH, CH)]],
                    gbuf.at[b], gsem.at[b])
        return carry

    lax.fori_loop(0, NG, ring, 0)


def _sc_base_body(gp, gn, ep, en, z80, accp_out, accn_out,
                  idx2, radja, gbuf, accp, accn, gsem, ssem):
    c = lax.axis_index("c")
    s = lax.axis_index("s")
    wid = s * NC + c
    pltpu.sync_copy(z80, accp.at[pl.ds(s * ZR, ZR)])
    pltpu.sync_copy(z80, accn.at[pl.ds(s * ZR, ZR)])
    plsc.subcore_barrier()

    _seg_sum_set(ep, gp, accp, wid, idx2, radja, gbuf, gsem, ssem)
    _seg_sum_set(en, gn, accn, wid, idx2, radja, gbuf, gsem, ssem)

    plsc.subcore_barrier()
    pltpu.sync_copy(accp.at[pl.ds(s * WR, WR)],
                    accp_out.at[c, pl.ds(s * WR, WR)])
    pltpu.sync_copy(accn.at[pl.ds(s * WR, WR)],
                    accn_out.at[c, pl.ds(s * WR, WR)])


def _sc_base(gp, gn, ep, en):
    z80 = jnp.zeros((ZR, FB), F32)
    mesh = plsc.VectorSubcoreMesh(core_axis_name="c", subcore_axis_name="s")
    fn = functools.partial(
        pl.kernel,
        mesh=mesh,
        out_type=[
            jax.ShapeDtypeStruct((NC, N, FB), F32),
            jax.ShapeDtypeStruct((NC, N, FB), F32),
        ],
        scratch_types=[
            pltpu.VMEM((2, EPW), jnp.int32),
            pltpu.VMEM((EPW,), jnp.int32),
            pltpu.VMEM((NB, CH, FB), F32),
            pltpu.VMEM_SHARED((NR, FB), F32),
            pltpu.VMEM_SHARED((NR, FB), F32),
            pltpu.SemaphoreType.DMA((NB,)),
            pltpu.SemaphoreType.DMA((NB,)),
        ],
    )(_sc_base_body)
    return fn(gp, gn, ep, en, z80)


def _combine_body(ap_ref, an_ref, ys_ref, bp_ref, bn_ref,
                  hcat_ref, inv_ref):
    ap = ap_ref[0] + ap_ref[1]
    an = an_ref[0] + an_ref[1]
    cp = ap[:, H:H + 1]
    cn = an[:, H:H + 1]
    hp = ap[:, :H] / jnp.maximum(cp, 1.0) + ys_ref[:, :H] + bp_ref[...]
    hn = an[:, :H] / jnp.maximum(cn, 1.0) + ys_ref[:, H:] + bn_ref[...]
    hp = jnp.tanh(_normalize_rows(hp))
    hn = jnp.tanh(_normalize_rows(hn))
    hcat_ref[...] = jnp.concatenate([hp, hn], axis=1)
    inv_ref[...] = jnp.concatenate(
        [jnp.broadcast_to(1.0 / (cp + 1.0), (BM, H)),
         jnp.broadcast_to(1.0 / (cn + 1.0), (BM, H))], axis=1)


def _combine(accp, accn, ys, bpb, bnb):
    grid = (N // BM,)
    return pl.pallas_call(
        _combine_body,
        grid=grid,
        in_specs=[
            pl.BlockSpec((NC, BM, FB), lambda i: (0, i, 0)),
            pl.BlockSpec((NC, BM, FB), lambda i: (0, i, 0)),
            pl.BlockSpec((BM, 2 * H), lambda i: (i, 0)),
            pl.BlockSpec((1, H), lambda i: (0, 0)),
            pl.BlockSpec((1, H), lambda i: (0, 0)),
        ],
        out_specs=[
            pl.BlockSpec((BM, FD), lambda i: (i, 0)),
            pl.BlockSpec((BM, FD), lambda i: (i, 0)),
        ],
        out_shape=[
            jax.ShapeDtypeStruct((N, FD), F32),
            jax.ShapeDtypeStruct((N, FD), F32),
        ],
    )(accp, accn, ys, bpb.reshape(1, H), bnb.reshape(1, H))


def _sc_deep_body(hcat, ep, en, z128, tp_out, tn_out,
                  idx2, radja, gbuf, accp, accn, gsem, ssem):
    c = lax.axis_index("c")
    s = lax.axis_index("s")
    wid = s * NC + c
    pltpu.sync_copy(z128, accp.at[pl.ds(s * ZR, ZR)])
    pltpu.sync_copy(z128, accn.at[pl.ds(s * ZR, ZR)])
    plsc.subcore_barrier()

    _seg_sum_set(ep, hcat, accp, wid, idx2, radja, gbuf, gsem, ssem)
    _seg_sum_set(en, hcat, accn, wid, idx2, radja, gbuf, gsem, ssem)

    plsc.subcore_barrier()
    pltpu.sync_copy(accp.at[pl.ds(s * WR, WR)],
                    tp_out.at[c, pl.ds(s * WR, WR)])
    pltpu.sync_copy(accn.at[pl.ds(s * WR, WR)],
                    tn_out.at[c, pl.ds(s * WR, WR)])


def _sc_deep(hcat, ep, en):
    z128 = jnp.zeros((ZR, FD), F32)
    mesh = plsc.VectorSubcoreMesh(core_axis_name="c", subcore_axis_name="s")
    fn = functools.partial(
        pl.kernel,
        mesh=mesh,
        out_type=[
            jax.ShapeDtypeStruct((NC, N, FD), F32),
            jax.ShapeDtypeStruct((NC, N, FD), F32),
        ],
        scratch_types=[
            pltpu.VMEM((2, EPW), jnp.int32),
            pltpu.VMEM((EPW,), jnp.int32),
            pltpu.VMEM((NB, CH, FD), F32),
            pltpu.VMEM_SHARED((NR, FD), F32),
            pltpu.VMEM_SHARED((NR, FD), F32),
            pltpu.SemaphoreType.DMA((NB,)),
            pltpu.SemaphoreType.DMA((NB,)),
        ],
    )(_sc_deep_body)
    return fn(hcat, ep, en, z128)


def _deep_combine_body(tp_ref, tn_ref, hcat_ref, inv_ref,
                       wp_ref, wn_ref, bp_ref, bn_ref, xmol_ref):
    hcat = hcat_ref[...]
    ip = inv_ref[:, :H]
    iv = inv_ref[:, H:]
    tph = tp_ref[0] + tp_ref[1] + hcat
    tnh = tn_ref[0] + tn_ref[1] + hcat
    hp0 = hcat[:, :H]
    hn0 = hcat[:, H:]
    catp = jnp.concatenate([tph[:, :H] * ip, tnh[:, H:] * iv, hp0], axis=1)
    catn = jnp.concatenate([tph[:, H:] * ip, tnh[:, :H] * iv, hn0], axis=1)
    hp_pre = jnp.dot(catp, wp_ref[...], preferred_element_type=F32) + bp_ref[...]
    hn_pre = jnp.dot(catn, wn_ref[...], preferred_element_type=F32) + bn_ref[...]
    hp1 = jnp.tanh(_normalize_rows(hp_pre))
    hn1 = jnp.tanh(_normalize_rows(hn_pre))
    xmol_ref[...] = _normalize_rows(jnp.concatenate([hp1, hn1], axis=1))


def _deep_combine(tp, tn, hcat, inv, Wpd, Wnd, bpd, bnd):
    grid = (N // BM,)
    return pl.pallas_call(
        _deep_combine_body,
        grid=grid,
        in_specs=[
            pl.BlockSpec((NC, BM, FD), lambda i: (0, i, 0)),
            pl.BlockSpec((NC, BM, FD), lambda i: (0, i, 0)),
            pl.BlockSpec((BM, FD), lambda i: (i, 0)),
            pl.BlockSpec((BM, FD), lambda i: (i, 0)),
            pl.BlockSpec((3 * H, H), lambda i: (0, 0)),
            pl.BlockSpec((3 * H, H), lambda i: (0, 0)),
            pl.BlockSpec((1, H), lambda i: (0, 0)),
            pl.BlockSpec((1, H), lambda i: (0, 0)),
        ],
        out_specs=pl.BlockSpec((BM, FD), lambda i: (i, 0)),
        out_shape=jax.ShapeDtypeStruct((N, FD), F32),
    )(tp, tn, hcat, inv, Wpd, Wnd,
      bpd.reshape(1, H), bnd.reshape(1, H))


BP = 512
GN_ = N // BP
RPB = BP * N // 128


def _pred_body(xi_ref, xall_ref, mask_ref, lab_ref, pred_ref, loss_ref):
    i = pl.program_id(0)

    @pl.when(i == 0)
    def _init():
        loss_ref[...] = jnp.zeros((1, 1), F32)

    b = lax.dot_general(xi_ref[...], xall_ref[...],
                        (((1,), (1,)), ((), ())),
                        preferred_element_type=F32) * mask_ref[...]
    b8 = b.reshape(RPB, 128)
    pred_ref[...] = b8
    r = b8 - lab_ref[...]
    loss_ref[...] += jnp.sum(r * r).reshape(1, 1)

    @pl.when(i == GN_ - 1)
    def _fin():
        loss_ref[...] = loss_ref[...] * (1.0 / float(N * N))


def _pred_loss(xmol, label_mask, labels8):
    grid = (GN_,)
    return pl.pallas_call(
        _pred_body,
        grid=grid,
        in_specs=[
            pl.BlockSpec((BP, FD), lambda i: (i, 0)),
            pl.BlockSpec((N, FD), lambda i: (0, 0)),
            pl.BlockSpec((BP, N), lambda i: (i, 0)),
            pl.BlockSpec((RPB, 128), lambda i: (i, 0)),
        ],
        out_specs=[
            pl.BlockSpec((RPB, 128), lambda i: (i, 0)),
            pl.BlockSpec((1, 1), lambda i: (0, 0)),
        ],
        out_shape=[
            jax.ShapeDtypeStruct((N * N // 128, 128), F32),
            jax.ShapeDtypeStruct((1, 1), F32),
        ],
    )(xmol, xmol, label_mask, labels8)


def kernel(X, positive_edges, negative_edges, labels, label_mask,
           Wpb, bpb, Wnb, bnb, Wpd, bpd, Wnd, bnd):
    ep = positive_edges.astype(jnp.int32)
    en = negative_edges.astype(jnp.int32)
    Wcat = jnp.concatenate([Wpb[:D], Wnb[:D], Wpb[D:], Wnb[D:]], axis=1)

    gp, gn, ys = _project(X, Wcat)
    accp, accn = _sc_base(gp, gn, ep, en)
    hcat, inv = _combine(accp, accn, ys, bpb, bnb)
    tp, tn = _sc_deep(hcat, ep, en)
    xmol = _deep_combine(tp, tn, hcat, inv, Wpd, Wnd, bpd, bnd)
    pred2, lossm = _pred_loss(xmol, label_mask, labels.reshape(N * N // 128, 128))
    return (lossm[0, 0], xmol, pred2.reshape(-1))
